# baseline (speedup 1.0000x reference)
import jax
import jax.numpy as jnp
from jax.experimental import pallas as pl
from jax.experimental.pallas import tpu as pltpu

_NUM_CG = 3
_TAU = 16
_NUM_SPECIES = 5
_CHARGE_POWER = 2
_NUM_BASIS = 10
_CHARGE_SCALE = 9.0
_SOFT_CUT_RAD = 1.73
_SOFT_CUT_WIDTH = 0.2
_N = 64
_NH = _N // 2
_NSI = _NUM_SPECIES * (_CHARGE_POWER + 1)
_KPAD = 16
_G = 8
_L = _G * _KPAD
_HID = 64
_B = 32
_WOFF = 6 * _B
_W1OFF = _WOFF + 13 * _TAU
_W2OFF = _W1OFF + 144
_B1OFF = _W2OFF + 1
_B2OFF = _B1OFF + 1
_PKROWS = 552
_RA = _NH * _N
_RB = _NH * _NH
_RT = _RA + _RB


def _edge_kernel(pk_ref, out_ref, wblk_s, w1_s, w2_s, sel_s):
    N, NH, L, G = _N, _NH, _L, _G
    f32 = jnp.float32
    gid = pl.program_id(0)

    @pl.when(pl.program_id(0) == 0)
    def _build_weights():
        rr = jax.lax.broadcasted_iota(jnp.int32, (L, L), 0)
        cc = jax.lax.broadcasted_iota(jnp.int32, (L, L), 1)
        blkmask = (rr // _TAU == cc // _TAU).astype(f32)
        for k in range(13):
            wk = pk_ref[_WOFF + k * _TAU: _WOFF + (k + 1) * _TAU, 0:_TAU]
            wk8 = jnp.concatenate([wk] * G, axis=0)
            wblk_s[k] = jnp.concatenate([wk8] * G, axis=1) * blkmask

        rr1 = jax.lax.broadcasted_iota(jnp.int32, (L, G * _HID), 0)
        cc1 = jax.lax.broadcasted_iota(jnp.int32, (L, G * _HID), 1)
        mask1 = (rr1 // _TAU == cc1 // _HID).astype(f32)
        sl = (2 + 1) * _TAU
        for lvl in range(_NUM_CG):
            w1l = pk_ref[_W1OFF + lvl * sl: _W1OFF + lvl * sl + _TAU, :]
            w1l8 = jnp.concatenate([w1l] * G, axis=0)
            w1_s[lvl * L:(lvl + 1) * L, :] = (
                jnp.concatenate([w1l8] * G, axis=1) * mask1)

        rr2 = jax.lax.broadcasted_iota(jnp.int32, (G * _HID, G), 0)
        cc2 = jax.lax.broadcasted_iota(jnp.int32, (G * _HID, G), 1)
        mask2 = (rr2 // _HID == cc2).astype(f32)
        w2col = jnp.swapaxes(pk_ref[_W2OFF:_W2OFF + 1, :], 0, 1)
        w2c = jnp.broadcast_to(w2col, (_HID, G))
        w2_s[...] = jnp.concatenate([w2c] * G, axis=0) * mask2

        rr3 = jax.lax.broadcasted_iota(jnp.int32, (L, G), 0)
        cc3 = jax.lax.broadcasted_iota(jnp.int32, (L, G), 1)
        sel_s[...] = (rr3 // _TAU == cc3).astype(f32) * (1.0 / _TAU)

    def chan(ch):
        blk = pk_ref[pl.ds(ch * _B + gid * G, G), :]
        t = jnp.swapaxes(blk, 0, 1)
        return jnp.concatenate(
            [jnp.broadcast_to(t[:, b:b + 1], (N, _KPAD)) for b in range(G)],
            axis=1)

    px = chan(0)
    py = chan(1)
    pz = chan(2)
    spf = chan(3)
    chg = chan(4)
    amg = chan(5)

    def pair(top, allv, sub):
        a = top[:, None, :] - allv[None, :, :]
        b = sub[:, None, :] - sub[None, :, :]
        return a.reshape(_RA, L), b.reshape(_RB, L)

    dxA, dxB = pair(px[:NH], px, px[NH:])
    dyA, dyB = pair(py[:NH], py, py[NH:])
    dzA, dzB = pair(pz[:NH], pz, pz[NH:])
    dx = jnp.concatenate([dxA, dxB], axis=0)
    dy = jnp.concatenate([dyA, dyB], axis=0)
    dz = jnp.concatenate([dzA, dzB], axis=0)
    dist2 = dx * dx + dy * dy + dz * dz
    norms = jnp.sqrt(jnp.maximum(dist2, 1e-12))

    iiA = jax.lax.broadcasted_iota(jnp.int32, (NH, N, 1), 0)
    jjA = jax.lax.broadcasted_iota(jnp.int32, (NH, N, 1), 1)
    odA = (iiA != jjA).astype(f32).reshape(_RA, 1)
    iiB = jax.lax.broadcasted_iota(jnp.int32, (NH, NH, 1), 0)
    jjB = jax.lax.broadcasted_iota(jnp.int32, (NH, NH, 1), 1)
    odB = (iiB != jjB).astype(f32).reshape(_RB, 1)
    off_diag = jnp.concatenate([odA, odB], axis=0)

    emA = (amg[:NH, None, :] * amg[None, :, :]).reshape(_RA, L)
    emB = (amg[NH:, None, :] * amg[NH:][None, :, :]).reshape(_RB, L)
    emask = jnp.concatenate([emA, emB], axis=0) * off_diag

    cut_f = (jax.nn.sigmoid((_SOFT_CUT_RAD - norms)
                            * (1.0 / _SOFT_CUT_WIDTH)) * emask)

    lane3 = jax.lax.broadcasted_iota(jnp.int32, (_RT, L), 1)
    ctr = (lane3 % _KPAD).astype(f32) * (4.0 / 9.0)
    dctr = norms - ctr
    basis_f = jnp.exp(dctr * dctr * (-1.0 / (2.0 * 0.3 * 0.3)))

    lane2 = jax.lax.broadcasted_iota(jnp.int32, (N, L), 1) % _KPAD
    onehot = ((spf == (lane2 // (_CHARGE_POWER + 1)).astype(f32))
              & (lane2 < _NSI)).astype(f32)
    c = chg * (1.0 / _CHARGE_SCALE)
    p = lane2 % (_CHARGE_POWER + 1)
    cpow = jnp.where(p == 0, 1.0, jnp.where(p == 1, c, c * c))
    scal = onehot * cpow * amg

    a = jnp.dot(scal, wblk_s[0], preferred_element_type=f32)

    e_list = []
    e_prev = None
    for lvl in range(_NUM_CG):
        rad = jnp.dot(basis_f, wblk_s[1 + lvl],
                      preferred_element_type=f32)
        dotsA = (a[:NH, None, :] * a[None, :, :]).reshape(_RA, L)
        dotsB = (a[NH:, None, :] * a[NH:][None, :, :]).reshape(_RB, L)
        dots = jnp.concatenate([dotsA, dotsB], axis=0)
        if e_prev is None:
            pre = dots
        else:
            pre = dots + jnp.dot(e_prev, wblk_s[4 + lvl],
                                 preferred_element_type=f32)
        e = pre * rad * cut_f
        eA3 = e[:_RA].reshape(NH, N, L)
        eB3 = e[_RA:].reshape(NH, NH, L)
        msg_top = jnp.sum(eA3, axis=1)
        msg_bot = jnp.sum(eA3[:, NH:, :], axis=0) + jnp.sum(eB3, axis=1)
        msg = jnp.concatenate([msg_top, msg_bot], axis=0)
        a = (jnp.dot(a, wblk_s[7 + lvl], preferred_element_type=f32)
             + jnp.dot(msg, wblk_s[10 + lvl],
                       preferred_element_type=f32)) * amg
        e_list.append(e)
        e_prev = e

    feat = jnp.concatenate(e_list, axis=1)
    h = jnp.dot(feat, w1_s[...], preferred_element_type=f32)
    b1u = pk_ref[_B1OFF:_B1OFF + 1, 0:_HID]
    b1row = jnp.concatenate([b1u] * G, axis=1)
    h = h + b1row
    h = jnp.maximum(h, 0.01 * h)
    pred = (jnp.dot(h, w2_s[...], preferred_element_type=f32)
            + pk_ref[_B2OFF:_B2OFF + 1, 0:1])

    em8 = jnp.dot(emask, sel_s[...],
                  preferred_element_type=f32)
    pred = pred * em8

    predA3 = pred[:_RA].reshape(NH, N, G)
    predB3 = pred[_RA:].reshape(NH, NH, G)
    q3 = jnp.swapaxes(predA3[:, NH:, :], 0, 1)
    bottom = jnp.concatenate([q3, predB3], axis=1)
    full_pred = jnp.concatenate([predA3, bottom], axis=0)
    out_ref[...] = full_pred.reshape(1, N, N, G)


def kernel(positions, species, charges, atom_mask,
           W_in, W_rad, W_prev, W_self, W_msg,
           W_top1, b_top1, W_top2, b_top2):
    B, N = positions.shape[0], positions.shape[1]
    T, G, L = _TAU, _G, _L
    NB = B // G
    f32 = jnp.float32

    amf = atom_mask.astype(f32)
    z1 = jnp.zeros((1, T), f32)
    z3 = jnp.zeros((_NUM_CG, _KPAD - _NUM_BASIS, T), f32)
    W_all = jnp.concatenate(
        [jnp.concatenate([W_in.astype(f32), z1], axis=0)[None],
         jnp.concatenate([W_rad.astype(f32), z3], axis=1),
         W_prev.astype(f32),
         W_self[:, 0].astype(f32),
         W_msg[:, 0].astype(f32)], axis=0).reshape(13 * T, T)
    pk = jnp.concatenate([
        positions[..., 0].astype(f32), positions[..., 1].astype(f32),
        positions[..., 2].astype(f32), species.astype(f32),
        charges.astype(f32), amf,
        jnp.concatenate([W_all, jnp.zeros((13 * T, N - T), f32)], axis=1),
        W_top1.astype(f32),
        W_top2.astype(f32).T,
        b_top1.astype(f32)[None, :],
        jnp.concatenate([b_top2.astype(f32),
                         jnp.zeros(N - 1, f32)])[None, :],
        jnp.zeros((_PKROWS - _B2OFF - 1, N), f32),
    ], axis=0)

    full = lambda shape: pl.BlockSpec(shape, lambda b: (0,) * len(shape))

    out = pl.pallas_call(
        _edge_kernel,
        grid=(NB,),
        in_specs=[
            full((_PKROWS, N)),
        ],
        out_specs=pl.BlockSpec((1, N, N, G), lambda b: (b, 0, 0, 0)),
        out_shape=jax.ShapeDtypeStruct((NB, N, N, G), f32),
        scratch_shapes=[
            pltpu.VMEM((13, L, L), f32),
            pltpu.VMEM((_NUM_CG * L, G * _HID), f32),
            pltpu.VMEM((G * _HID, G), f32),
            pltpu.VMEM((L, G), f32),
        ],
        compiler_params=pltpu.CompilerParams(
            dimension_semantics=("arbitrary",)),
    )(pk)

    return out.transpose(0, 3, 1, 2).reshape(B, N, N, 1)


# speedup vs baseline: 1.0212x; 1.0212x over previous
import jax
import jax.numpy as jnp
from jax.experimental import pallas as pl
from jax.experimental.pallas import tpu as pltpu

_NUM_CG = 3
_TAU = 16
_NUM_SPECIES = 5
_CHARGE_POWER = 2
_NUM_BASIS = 10
_CHARGE_SCALE = 9.0
_SOFT_CUT_RAD = 1.73
_SOFT_CUT_WIDTH = 0.2
_N = 64
_NH = _N // 2
_NSI = _NUM_SPECIES * (_CHARGE_POWER + 1)
_KPAD = 16
_G = 8
_L = _G * _KPAD
_HID = 64
_B = 32
_RA = _NH * _N
_RB = _NH * _NH
_RT = _RA + _RB


def _edge_kernel(chans_ref, wall_ref, w1raw_ref, w2raw_ref, bz_ref,
                 out_ref, wblk_s, w1_s, w2_s, sel_s):
    N, NH, L, G = _N, _NH, _L, _G
    f32 = jnp.float32
    gid = pl.program_id(0)

    @pl.when(pl.program_id(0) == 0)
    def _build_weights():
        rr = jax.lax.broadcasted_iota(jnp.int32, (L, L), 0)
        cc = jax.lax.broadcasted_iota(jnp.int32, (L, L), 1)
        blkmask = (rr // _TAU == cc // _TAU).astype(f32)
        for k in range(13):
            wk = wall_ref[k]
            wk8 = jnp.concatenate([wk] * G, axis=0)
            wblk_s[k] = jnp.concatenate([wk8] * G, axis=1) * blkmask

        rr1 = jax.lax.broadcasted_iota(jnp.int32, (L, G * _HID), 0)
        cc1 = jax.lax.broadcasted_iota(jnp.int32, (L, G * _HID), 1)
        mask1 = (rr1 // _TAU == cc1 // _HID).astype(f32)
        sl = (2 + 1) * _TAU
        for lvl in range(_NUM_CG):
            w1l = w1raw_ref[lvl * sl: lvl * sl + _TAU, :]
            w1l8 = jnp.concatenate([w1l] * G, axis=0)
            w1_s[lvl * L:(lvl + 1) * L, :] = (
                jnp.concatenate([w1l8] * G, axis=1) * mask1)

        rr2 = jax.lax.broadcasted_iota(jnp.int32, (G * _HID, G), 0)
        cc2 = jax.lax.broadcasted_iota(jnp.int32, (G * _HID, G), 1)
        mask2 = (rr2 // _HID == cc2).astype(f32)
        w2c = jnp.broadcast_to(w2raw_ref[...], (_HID, G))
        w2_s[...] = jnp.concatenate([w2c] * G, axis=0) * mask2

        rr3 = jax.lax.broadcasted_iota(jnp.int32, (L, G), 0)
        cc3 = jax.lax.broadcasted_iota(jnp.int32, (L, G), 1)
        sel_s[...] = (rr3 // _TAU == cc3).astype(f32) * (1.0 / _TAU)

    def chan(ch):
        blk = chans_ref[pl.ds(ch * _B + gid * G, G), :]
        t = jnp.swapaxes(blk, 0, 1)
        return jnp.concatenate(
            [jnp.broadcast_to(t[:, b:b + 1], (N, _KPAD)) for b in range(G)],
            axis=1)

    px = chan(0)
    py = chan(1)
    pz = chan(2)
    spf = chan(3)
    chg = chan(4)
    amg = chan(5)

    def pair(top, allv, sub):
        a = top[:, None, :] - allv[None, :, :]
        b = sub[:, None, :] - sub[None, :, :]
        return a.reshape(_RA, L), b.reshape(_RB, L)

    dxA, dxB = pair(px[:NH], px, px[NH:])
    dyA, dyB = pair(py[:NH], py, py[NH:])
    dzA, dzB = pair(pz[:NH], pz, pz[NH:])
    dx = jnp.concatenate([dxA, dxB], axis=0)
    dy = jnp.concatenate([dyA, dyB], axis=0)
    dz = jnp.concatenate([dzA, dzB], axis=0)
    dist2 = dx * dx + dy * dy + dz * dz
    norms = jnp.sqrt(jnp.maximum(dist2, 1e-12))

    iiA = jax.lax.broadcasted_iota(jnp.int32, (NH, N, 1), 0)
    jjA = jax.lax.broadcasted_iota(jnp.int32, (NH, N, 1), 1)
    odA = (iiA != jjA).astype(f32).reshape(_RA, 1)
    iiB = jax.lax.broadcasted_iota(jnp.int32, (NH, NH, 1), 0)
    jjB = jax.lax.broadcasted_iota(jnp.int32, (NH, NH, 1), 1)
    odB = (iiB != jjB).astype(f32).reshape(_RB, 1)
    off_diag = jnp.concatenate([odA, odB], axis=0)

    emA = (amg[:NH, None, :] * amg[None, :, :]).reshape(_RA, L)
    emB = (amg[NH:, None, :] * amg[NH:][None, :, :]).reshape(_RB, L)
    emask = jnp.concatenate([emA, emB], axis=0) * off_diag

    cut_f = (jax.nn.sigmoid((_SOFT_CUT_RAD - norms)
                            * (1.0 / _SOFT_CUT_WIDTH)) * emask)

    lane3 = jax.lax.broadcasted_iota(jnp.int32, (_RT, L), 1)
    ctr = (lane3 % _KPAD).astype(f32) * (4.0 / 9.0)
    dctr = norms - ctr
    basis_f = jnp.exp(dctr * dctr * (-1.0 / (2.0 * 0.3 * 0.3)))

    lane2 = jax.lax.broadcasted_iota(jnp.int32, (N, L), 1) % _KPAD
    onehot = ((spf == (lane2 // (_CHARGE_POWER + 1)).astype(f32))
              & (lane2 < _NSI)).astype(f32)
    c = chg * (1.0 / _CHARGE_SCALE)
    p = lane2 % (_CHARGE_POWER + 1)
    cpow = jnp.where(p == 0, 1.0, jnp.where(p == 1, c, c * c))
    scal = onehot * cpow * amg

    a = jnp.dot(scal, wblk_s[0], preferred_element_type=f32)

    e_list = []
    e_prev = None
    for lvl in range(_NUM_CG):
        rad = jnp.dot(basis_f, wblk_s[1 + lvl],
                      preferred_element_type=f32)
        dotsA = (a[:NH, None, :] * a[None, :, :]).reshape(_RA, L)
        dotsB = (a[NH:, None, :] * a[NH:][None, :, :]).reshape(_RB, L)
        dots = jnp.concatenate([dotsA, dotsB], axis=0)
        if e_prev is None:
            pre = dots
        else:
            pre = dots + jnp.dot(e_prev, wblk_s[4 + lvl],
                                 preferred_element_type=f32)
        e = pre * rad * cut_f
        eA3 = e[:_RA].reshape(NH, N, L)
        eB3 = e[_RA:].reshape(NH, NH, L)
        msg_top = jnp.sum(eA3, axis=1)
        msg_bot = jnp.sum(eA3[:, NH:, :], axis=0) + jnp.sum(eB3, axis=1)
        msg = jnp.concatenate([msg_top, msg_bot], axis=0)
        a = (jnp.dot(a, wblk_s[7 + lvl], preferred_element_type=f32)
             + jnp.dot(msg, wblk_s[10 + lvl],
                       preferred_element_type=f32)) * amg
        e_list.append(e)
        e_prev = e

    feat = jnp.concatenate(e_list, axis=1)
    h = jnp.dot(feat, w1_s[...], preferred_element_type=f32)
    b1u = bz_ref[0:1, 0:_HID]
    b1row = jnp.concatenate([b1u] * G, axis=1)
    h = h + b1row
    h = jnp.maximum(h, 0.01 * h)
    pred = (jnp.dot(h, w2_s[...], preferred_element_type=f32)
            + bz_ref[0:1, _HID:_HID + 1])

    em8 = jnp.dot(emask, sel_s[...],
                  preferred_element_type=f32)
    pred = pred * em8

    predA3 = pred[:_RA].reshape(NH, N, G)
    predB3 = pred[_RA:].reshape(NH, NH, G)
    q3 = jnp.swapaxes(predA3[:, NH:, :], 0, 1)
    bottom = jnp.concatenate([q3, predB3], axis=1)
    full_pred = jnp.concatenate([predA3, bottom], axis=0)
    out_ref[...] = full_pred.reshape(1, N, N, G)


def kernel(positions, species, charges, atom_mask,
           W_in, W_rad, W_prev, W_self, W_msg,
           W_top1, b_top1, W_top2, b_top2):
    B, N = positions.shape[0], positions.shape[1]
    T, G, L = _TAU, _G, _L
    NB = B // G
    f32 = jnp.float32

    amf = atom_mask.astype(f32)
    chans = jnp.stack([positions[..., 0].astype(f32),
                       positions[..., 1].astype(f32),
                       positions[..., 2].astype(f32),
                       species.astype(f32),
                       charges.astype(f32),
                       amf], axis=0).reshape(6 * B, N)

    z1 = jnp.zeros((1, T), f32)
    z3 = jnp.zeros((_NUM_CG, _KPAD - _NUM_BASIS, T), f32)
    W_all = jnp.concatenate(
        [jnp.concatenate([W_in.astype(f32), z1], axis=0)[None],
         jnp.concatenate([W_rad.astype(f32), z3], axis=1),
         W_prev.astype(f32),
         W_self[:, 0].astype(f32),
         W_msg[:, 0].astype(f32)], axis=0)

    bz = jnp.broadcast_to(
        jnp.concatenate([b_top1.astype(f32), b_top2.astype(f32),
                         jnp.zeros(128 - _HID - 1, f32)])[None, :], (8, 128))

    full = lambda shape: pl.BlockSpec(shape, lambda b: (0,) * len(shape))

    out = pl.pallas_call(
        _edge_kernel,
        grid=(NB,),
        in_specs=[
            full((6 * B, N)),
            full((13, T, T)),
            full((_NUM_CG * (2 + 1) * T, _HID)),
            full((_HID, 1)),
            full((8, 128)),
        ],
        out_specs=pl.BlockSpec((1, N, N, G), lambda b: (b, 0, 0, 0)),
        out_shape=jax.ShapeDtypeStruct((NB, N, N, G), f32),
        scratch_shapes=[
            pltpu.VMEM((13, L, L), f32),
            pltpu.VMEM((_NUM_CG * L, G * _HID), f32),
            pltpu.VMEM((G * _HID, G), f32),
            pltpu.VMEM((L, G), f32),
        ],
        compiler_params=pltpu.CompilerParams(
            dimension_semantics=("arbitrary",)),
    )(chans, W_all, W_top1.astype(f32), W_top2.astype(f32), bz)

    return out.transpose(0, 3, 1, 2).reshape(B, N, N, 1)
